# baseline (speedup 1.0000x reference)
import jax
import jax.numpy as jnp
from jax import lax
from jax.experimental import pallas as pl
from jax.experimental.pallas import tpu as pltpu
from jax.experimental.pallas import tpu_sc as plsc

F = 26
NV = 100000
B = 16384
NC = 2
NS = 16
NW = NC * NS
BPW = B // NW
CHUNK = BPW * F
SPF = BPW // 16
RG = BPW // 16


def _sc_body(xt_hbm, table_hbm, bias_hbm, out_hbm, x_v, idx_v, g_v, out_v,
             bias_v, sem, gsem):
    c = lax.axis_index("c")
    s = lax.axis_index("s")
    wid = s * NC + c
    base = wid * BPW

    copies = [
        pltpu.make_async_copy(
            xt_hbm.at[pl.ds(f * B + base, BPW)],
            x_v.at[pl.ds(f * BPW, BPW)],
            sem,
        )
        for f in range(F)
    ]
    for cp in copies:
        cp.start()
    pltpu.sync_copy(bias_hbm, bias_v)

    gathers = [
        pltpu.make_async_copy(
            table_hbm.at[idx_v.at[pl.ds(f * BPW, BPW)]],
            g_v.at[pl.ds(f * BPW, BPW)],
            gsem,
        )
        for f in range(F)
    ]
    for f in range(F):
        copies[f].wait()

        def add_off(i, carry, f=f):
            j = f * SPF + i
            idx_v[pl.ds(j * 16, 16)] = x_v[pl.ds(j * 16, 16)] + (f * NV)
            return carry

        lax.fori_loop(0, SPF, add_off, 0)
        gathers[f].start()
    for g in gathers:
        g.wait()

    bias16 = bias_v[...]

    def reduce_rows(rg, carry):
        r0 = rg * 16
        acc = g_v[pl.ds(r0, 16)]
        for f in range(1, F):
            acc = acc + g_v[pl.ds(f * BPW + r0, 16)]
        out_v[pl.ds(r0, 16)] = acc + bias16
        return carry

    lax.fori_loop(0, RG, reduce_rows, 0)

    pltpu.sync_copy(out_v, out_hbm.at[pl.ds(base, BPW)])


@jax.jit
def kernel(x, table, bias):
    xt = x.T.reshape(-1)
    t2 = jax.lax.optimization_barrier(table.reshape(40625, 64))
    tf = t2.reshape(-1)
    b16 = jnp.broadcast_to(bias, (16,)).astype(jnp.float32)

    mesh = plsc.VectorSubcoreMesh(core_axis_name="c", subcore_axis_name="s")
    run = pl.kernel(
        _sc_body,
        out_type=jax.ShapeDtypeStruct((B,), jnp.float32),
        mesh=mesh,
        scratch_types=[
            pltpu.VMEM((CHUNK,), jnp.int32),
            pltpu.VMEM((CHUNK,), jnp.int32),
            pltpu.VMEM((CHUNK,), jnp.float32),
            pltpu.VMEM((BPW,), jnp.float32),
            pltpu.VMEM((16,), jnp.float32),
            pltpu.SemaphoreType.DMA,
            pltpu.SemaphoreType.DMA,
        ],
    )
    out = run(xt, tf, b16)
    return out.reshape(B, 1)


# speedup vs baseline: 1.2177x; 1.2177x over previous
import jax
import jax.numpy as jnp
from jax import lax
from jax.experimental import pallas as pl
from jax.experimental.pallas import tpu as pltpu
from jax.experimental.pallas import tpu_sc as plsc

F = 26
NV = 100000
B = 16384
NC = 2
NS = 16
NW = NC * NS
BPW = B // NW
CHUNK = BPW * F
SPF = BPW // 16
RG = BPW // 16


def _sc_body(xt_hbm, table_hbm, bias_hbm, out_hbm, x_v, idx_v, g_v, out_v,
             bias_v, sem, gsem):
    c = lax.axis_index("c")
    s = lax.axis_index("s")
    wid = s * NC + c
    base = wid * BPW

    copies = [
        pltpu.make_async_copy(
            xt_hbm.at[pl.ds(f * B + base, BPW)],
            x_v.at[pl.ds(f * BPW, BPW)],
            sem,
        )
        for f in range(F)
    ]
    for cp in copies:
        cp.start()
    pltpu.sync_copy(bias_hbm, bias_v)

    gathers = [
        pltpu.make_async_copy(
            table_hbm.at[idx_v.at[pl.ds(f * BPW, BPW)]],
            g_v.at[pl.ds(f * BPW, BPW)],
            gsem,
        )
        for f in range(F)
    ]
    for f in range(F):
        copies[f].wait()

        def add_off(i, carry, f=f):
            j = f * SPF + i
            idx_v[pl.ds(j * 16, 16)] = x_v[pl.ds(j * 16, 16)] + (f * NV)
            return carry

        lax.fori_loop(0, SPF, add_off, 0)
        gathers[f].start()
    for g in gathers:
        g.wait()

    bias16 = bias_v[...]

    def reduce_rows(rg, carry):
        r0 = rg * 16
        acc = g_v[pl.ds(r0, 16)]
        for f in range(1, F):
            acc = acc + g_v[pl.ds(f * BPW + r0, 16)]
        out_v[pl.ds(r0, 16)] = acc + bias16
        return carry

    lax.fori_loop(0, RG, reduce_rows, 0)

    pltpu.sync_copy(out_v, out_hbm.at[pl.ds(base, BPW)])


@jax.jit
def kernel(x, table, bias):
    xt = x.T.reshape(-1)
    t2 = jax.lax.optimization_barrier(table.reshape(1, 2600000))
    tf = t2.reshape(-1)
    b16 = jnp.broadcast_to(bias, (16,)).astype(jnp.float32)

    mesh = plsc.VectorSubcoreMesh(core_axis_name="c", subcore_axis_name="s")
    run = pl.kernel(
        _sc_body,
        out_type=jax.ShapeDtypeStruct((B,), jnp.float32),
        mesh=mesh,
        scratch_types=[
            pltpu.VMEM((CHUNK,), jnp.int32),
            pltpu.VMEM((CHUNK,), jnp.int32),
            pltpu.VMEM((CHUNK,), jnp.float32),
            pltpu.VMEM((BPW,), jnp.float32),
            pltpu.VMEM((16,), jnp.float32),
            pltpu.SemaphoreType.DMA,
            pltpu.SemaphoreType.DMA,
        ],
    )
    out = run(xt, tf, b16)
    return out.reshape(B, 1)


# speedup vs baseline: 1.2180x; 1.0003x over previous
import jax
import jax.numpy as jnp
from jax import lax
from jax.experimental import pallas as pl
from jax.experimental.pallas import tpu as pltpu
from jax.experimental.pallas import tpu_sc as plsc

F = 26
NV = 100000
B = 16384
NC = 2
NS = 16
NW = NC * NS
BPW = B // NW
CHUNK = BPW * F
SPF = BPW // 16
RG = BPW // 16


def _sc_body(xt_hbm, table_hbm, bias_hbm, out_hbm, x_v, idx_v, g_v, out_v,
             bias_v, sem, gsem):
    c = lax.axis_index("c")
    s = lax.axis_index("s")
    wid = s * NC + c
    base = wid * BPW

    copies = [
        pltpu.make_async_copy(
            xt_hbm.at[pl.ds(f * B + base, BPW)],
            x_v.at[pl.ds(f * BPW, BPW)],
            sem,
        )
        for f in range(F)
    ]
    for cp in copies:
        cp.start()
    pltpu.sync_copy(bias_hbm, bias_v)

    gathers = [
        pltpu.make_async_copy(
            table_hbm.at[idx_v.at[pl.ds(f * BPW, BPW)]],
            g_v.at[pl.ds(f * BPW, BPW)],
            gsem,
        )
        for f in range(F)
    ]
    for f in range(F):
        copies[f].wait()

        def add_off(i, carry, f=f):
            j = f * SPF + i
            idx_v[pl.ds(j * 16, 16)] = x_v[pl.ds(j * 16, 16)] + (f * NV)
            return carry

        lax.fori_loop(0, SPF, add_off, 0)
        gathers[f].start()
    HALF = 13
    for g in gathers[:HALF]:
        g.wait()

    bias16 = bias_v[...]

    def reduce_rows_lo(rg, carry):
        r0 = rg * 16
        acc = g_v[pl.ds(r0, 16)] + bias16
        for f in range(1, HALF):
            acc = acc + g_v[pl.ds(f * BPW + r0, 16)]
        out_v[pl.ds(r0, 16)] = acc
        return carry

    lax.fori_loop(0, RG, reduce_rows_lo, 0)

    for g in gathers[HALF:]:
        g.wait()

    def reduce_rows_hi(rg, carry):
        r0 = rg * 16
        acc = out_v[pl.ds(r0, 16)]
        for f in range(HALF, F):
            acc = acc + g_v[pl.ds(f * BPW + r0, 16)]
        out_v[pl.ds(r0, 16)] = acc
        return carry

    lax.fori_loop(0, RG, reduce_rows_hi, 0)

    pltpu.sync_copy(out_v, out_hbm.at[pl.ds(base, BPW)])


@jax.jit
def kernel(x, table, bias):
    xt = x.T.reshape(-1)
    tf = table.reshape(-1)
    b16 = jnp.broadcast_to(bias, (16,)).astype(jnp.float32)

    mesh = plsc.VectorSubcoreMesh(core_axis_name="c", subcore_axis_name="s")
    run = pl.kernel(
        _sc_body,
        out_type=jax.ShapeDtypeStruct((B,), jnp.float32),
        mesh=mesh,
        scratch_types=[
            pltpu.VMEM((CHUNK,), jnp.int32),
            pltpu.VMEM((CHUNK,), jnp.int32),
            pltpu.VMEM((CHUNK,), jnp.float32),
            pltpu.VMEM((BPW,), jnp.float32),
            pltpu.VMEM((16,), jnp.float32),
            pltpu.SemaphoreType.DMA,
            pltpu.SemaphoreType.DMA,
        ],
    )
    out = run(xt, tf, b16)
    return out.reshape(B, 1)
